# baseline (speedup 1.0000x reference)
import jax
import jax.numpy as jnp
from jax.experimental import pallas as pl
from jax.experimental.pallas import tpu as pltpu

M = 256
H = 128
ED = 128
ET = 4
EF = 4
NS = 57
BS = 16
NBLK = M // BS
CHUNK = 4096
NCH = (H * M) // CHUNK
ROWS = CHUNK // H


def _cf0_kernel(pf_ref, wp_ref, bp_ref, o_ref):
    x = jnp.dot(pf_ref[:, :], wp_ref[:, :], preferred_element_type=jnp.float32)
    o_ref[:, :] = jnp.maximum(x + bp_ref[:, :], 0.0)


NEG = -1e30


def _uv_prologue(cf, wex_ref, bex_ref, wel_ref, bel_ref,
                 u_scr, v_scr, cexl_scr, pend_scr):
    wel = wel_ref[:, :]
    u_scr[:, :] = jnp.dot(cf, wel[:H, :], preferred_element_type=jnp.float32) + bel_ref[:, :]
    v_scr[:, :] = jnp.dot(cf, wel[H:, :], preferred_element_type=jnp.float32)
    cexl_scr[:, :] = jnp.dot(cf, wex_ref[:, :], preferred_element_type=jnp.float32) + bex_ref[:, :]
    pend_scr[:, :] = _pend_col(cexl_scr)


def _masked_reduce(el2, logits2, rhs_ref, cexl_ref, wne, bne,
                   a_scr, b_scr, pend_scr, k):
    wel_i = wne[2 * H:3 * H, :]
    c = jnp.dot(el2, wel_i, preferred_element_type=jnp.float32).reshape(BS, M, H)
    maskpen = (jnp.where(logits2 > 0, jnp.float32(0.0), NEG)
               + pend_scr[:, :])
    lhs = jnp.concatenate([logits2, maskpen], axis=1)
    terms = jnp.dot(lhs, rhs_ref[:, :], preferred_element_type=jnp.float32)
    dmax = jnp.maximum(jnp.maximum(terms[:, :H], terms[:, H:2 * H]),
                       jnp.maximum(terms[:, 2 * H:3 * H], terms[:, 3 * H:]))
    scores = b_scr[:, :][None, :, :] + c + dmax.reshape(BS, M, H)
    d_red = jnp.max(scores, axis=1)
    cexs = cexl_ref[pl.ds(k * BS, BS), :]
    s01 = jnp.where(cexs > 0, jnp.float32(1.0), jnp.float32(0.0))
    a_blk = a_scr[pl.ds(k * BS, BS), :]
    ncf = jnp.maximum(a_blk + bne + d_red, 0.0) * s01
    valid4 = maskpen.reshape(BS, M, ET)
    rowvalid = jnp.max(valid4, axis=(1, 2), keepdims=True).reshape(BS, 1)
    rowflag = jnp.where(jnp.logical_and(rowvalid > NEG * 0.5, cexs > 0),
                        jnp.float32(1.0), jnp.float32(0.0))
    return ncf, rowflag


def _pend_col(cexl_ref):
    pend_m = jnp.where(cexl_ref[:, :] > 0, jnp.float32(0.0), NEG)
    return jnp.broadcast_to(pend_m[None, :, :], (BS, M, 1)).reshape(BS * M, 1)


def _iter1_kernel(wex_ref, bex_ref, wel_ref, bel_ref,
                  wee_ref, bee_ref, bee_t_ref, wef_ref, bef_t_ref,
                  wne_ref, bne_ref, rhs_ref, cf0_ref,
                  cexl_out_ref, eel_ref, ef_ref, ncf_ref, any_ref,
                  a_scr, b_scr, pend_scr, u_scr, v_scr, cexl_scr):
    k = pl.program_id(0)
    wne = wne_ref[:, :]

    @pl.when(k == 0)
    def _():
        cf = cf0_ref[:, :]
        _uv_prologue(cf, wex_ref, bex_ref, wel_ref, bel_ref,
                     u_scr, v_scr, cexl_scr, pend_scr)
        cexl_out_ref[:, :] = cexl_scr[:, :]
        a_scr[:, :] = jnp.dot(cf, wne[:H, :], preferred_element_type=jnp.float32)
        b_scr[:, :] = jnp.dot(cf, wne[H:2 * H, :], preferred_element_type=jnp.float32)
        any_ref[:, :] = jnp.zeros((1, 1), jnp.float32)

    el = jnp.maximum(u_scr[pl.ds(k * BS, BS), :][:, None, :]
                     + v_scr[:, :][None, :, :], 0.0)
    el2 = el.reshape(BS * M, ED)
    logits2 = jnp.dot(el2, wee_ref[:, :], preferred_element_type=jnp.float32) + bee_ref[:, :]
    eelt = jax.lax.dot_general(wee_ref[:, :], el2, (((0,), (1,)), ((), ())),
                               preferred_element_type=jnp.float32)
    eel_ref[:, :] = eelt + bee_t_ref[:, :]
    eft = jax.lax.dot_general(wef_ref[:, :], el2, (((0,), (1,)), ((), ())),
                              preferred_element_type=jnp.float32)
    ef_ref[:, :] = eft + bef_t_ref[:, :]

    ncf, rowflag = _masked_reduce(el2, logits2, rhs_ref, cexl_scr, wne,
                                  bne_ref[:, :], a_scr, b_scr, pend_scr, k)
    ncf_ref[:, :] = ncf
    anyblk = jnp.max(rowflag, axis=(0, 1), keepdims=True)
    any_ref[:, :] = jnp.maximum(any_ref[:, :], anyblk)


def _iter2_kernel(wex_ref, bex_ref, wel_ref, bel_ref, wee_ref, bee_ref,
                  wne_ref, bne_ref, rhs_ref,
                  wc_ref, bc_ref, ws_ref, bs_ref, wc2_ref, bc2_ref,
                  cf0_ref, ncf1_ref, any_ref,
                  sem_ref, out_ref,
                  a_scr, b_scr, pend_scr, u_scr, v_scr, cexl_scr, ncf2_scr):
    k = pl.program_id(0)
    wne = wne_ref[:, :]

    @pl.when(k == 0)
    def _():
        _uv_prologue(cf0_ref[:, :], wex_ref, bex_ref, wel_ref, bel_ref,
                     u_scr, v_scr, cexl_scr, pend_scr)
        anyf = any_ref[:, :]
        cf = ncf1_ref[:, :] * anyf + cf0_ref[:, :] * (1.0 - anyf)
        a_scr[:, :] = jnp.dot(cf, wne[:H, :], preferred_element_type=jnp.float32)
        b_scr[:, :] = jnp.dot(cf, wne[H:2 * H, :], preferred_element_type=jnp.float32)

    el = jnp.maximum(u_scr[pl.ds(k * BS, BS), :][:, None, :]
                     + v_scr[:, :][None, :, :], 0.0)
    el2 = el.reshape(BS * M, ED)
    logits2 = jnp.dot(el2, wee_ref[:, :], preferred_element_type=jnp.float32) + bee_ref[:, :]
    ncf, _ = _masked_reduce(el2, logits2, rhs_ref, cexl_scr, wne,
                            bne_ref[:, :], a_scr, b_scr, pend_scr, k)
    ncf2_scr[pl.ds(k * BS, BS), :] = ncf

    @pl.when(k == NBLK - 1)
    def _():
        anyf = any_ref[:, :]
        cf0 = cf0_ref[:, :]
        cf1 = ncf1_ref[:, :] * anyf + cf0 * (1.0 - anyf)
        cf2 = ncf2_scr[:, :] * anyf + cf0 * (1.0 - anyf)
        cat = jnp.concatenate([cf0, cf1, cf2], axis=1)
        h = jnp.maximum(jnp.dot(cat, wc_ref[:, :], preferred_element_type=jnp.float32)
                        + bc_ref[:, :], 0.0)
        sem_ref[:, :] = jnp.dot(h, ws_ref[:, :], preferred_element_type=jnp.float32) + bs_ref[:, :]
        out_ref[:, :] = jnp.maximum(
            jnp.dot(h, wc2_ref[:, :], preferred_element_type=jnp.float32) + bc2_ref[:, :], 0.0)


def _full(shape):
    return pl.BlockSpec(shape, lambda *_: tuple(0 for _ in shape))


def kernel(parent_feature, Wp, bp, Wex, bex, Wel, bel, Wee, bee, Wne, bne,
           Wc, bc, Ws, bs, Wc2, bc2, Wef, bef):
    f32 = jnp.float32
    bpR = bp.reshape(1, M * H)
    bexR = bex.reshape(1, 1)
    belR = bel.reshape(1, ED)
    WeeR = Wee.reshape(ET, ED).T
    beeR = bee.reshape(1, ET)
    WefR = jnp.transpose(Wef, (1, 0, 2)).reshape(ED, ET * EF)
    befR = bef.reshape(1, ET * EF)
    eye = jnp.eye(ET, dtype=f32)[:, :, None]
    rhs_list = []
    for i in range(2):
        wet = Wne[i, 3 * H:3 * H + ET, :]
        top = (eye * wet[None, :, :]).reshape(ET, ET * H)
        mid = jnp.broadcast_to(eye, (ET, ET, H)).reshape(ET, ET * H)
        rhs_list.append(jnp.concatenate([top, mid], axis=0))

    cf0 = pl.pallas_call(
        _cf0_kernel,
        grid=(NCH,),
        in_specs=[
            pl.BlockSpec((1, H), lambda j: (0, 0)),
            pl.BlockSpec((H, CHUNK), lambda j: (0, j)),
            pl.BlockSpec((1, CHUNK), lambda j: (0, j)),
        ],
        out_specs=pl.BlockSpec((1, CHUNK), lambda j: (0, j)),
        out_shape=jax.ShapeDtypeStruct((1, M * H), f32),
    )(parent_feature, Wp, bpR).reshape(M, H)

    cexl, eelF, efF, ncf1, anyv = pl.pallas_call(
        _iter1_kernel,
        grid=(NBLK,),
        in_specs=[
            _full((H, 1)),
            _full((1, 1)),
            _full((2 * H, ED)),
            _full((1, ED)),
            _full((ED, ET)),
            _full((1, ET)),
            _full((ET, 1)),
            _full((ED, ET * EF)),
            _full((ET * EF, 1)),
            _full((3 * H + ET, H)),
            _full((1, H)),
            _full((2 * ET, ET * H)),
            _full((M, H)),
        ],
        out_specs=[
            pl.BlockSpec((M, 1), lambda k: (0, 0)),
            pl.BlockSpec((ET, BS * M), lambda k: (0, k)),
            pl.BlockSpec((ET * EF, BS * M), lambda k: (0, k)),
            pl.BlockSpec((BS, H), lambda k: (k, 0)),
            pl.BlockSpec((1, 1), lambda k: (0, 0)),
        ],
        out_shape=[jax.ShapeDtypeStruct((M, 1), f32),
                   jax.ShapeDtypeStruct((ET, M * M), f32),
                   jax.ShapeDtypeStruct((ET * EF, M * M), f32),
                   jax.ShapeDtypeStruct((M, H), f32),
                   jax.ShapeDtypeStruct((1, 1), f32)],
        scratch_shapes=[pltpu.VMEM((M, H), f32), pltpu.VMEM((M, H), f32),
                        pltpu.VMEM((BS * M, 1), f32),
                        pltpu.VMEM((M, ED), f32), pltpu.VMEM((M, ED), f32),
                        pltpu.VMEM((M, 1), f32)],
    )(Wex, bexR, Wel, belR, WeeR, beeR, bee.reshape(ET, 1),
      WefR, bef.reshape(ET * EF, 1),
      Wne[0], bne[0].reshape(1, H), rhs_list[0], cf0)

    sem, child_out = pl.pallas_call(
        _iter2_kernel,
        grid=(NBLK,),
        in_specs=[
            _full((H, 1)),
            _full((1, 1)),
            _full((2 * H, ED)),
            _full((1, ED)),
            _full((ED, ET)),
            _full((1, ET)),
            _full((3 * H + ET, H)),
            _full((1, H)),
            _full((2 * ET, ET * H)),
            _full((3 * H, H)),
            _full((1, H)),
            _full((H, NS)),
            _full((1, NS)),
            _full((H, H)),
            _full((1, H)),
            _full((M, H)),
            _full((M, H)),
            _full((1, 1)),
        ],
        out_specs=[
            pl.BlockSpec((M, NS), lambda k: (0, 0)),
            pl.BlockSpec((M, H), lambda k: (0, 0)),
        ],
        out_shape=[jax.ShapeDtypeStruct((M, NS), f32),
                   jax.ShapeDtypeStruct((M, H), f32)],
        scratch_shapes=[pltpu.VMEM((M, H), f32), pltpu.VMEM((M, H), f32),
                        pltpu.VMEM((BS * M, 1), f32),
                        pltpu.VMEM((M, ED), f32), pltpu.VMEM((M, ED), f32),
                        pltpu.VMEM((M, 1), f32), pltpu.VMEM((M, H), f32)],
    )(Wex, bexR, Wel, belR, WeeR, beeR, Wne[1], bne[1].reshape(1, H),
      rhs_list[1], Wc, bc.reshape(1, H), Ws, bs.reshape(1, NS),
      Wc2, bc2.reshape(1, H), cf0, ncf1, anyv)

    eel_out = jnp.transpose(eelF.reshape(ET, M, M), (1, 2, 0))
    ef_out = jnp.transpose(efF.reshape(ET, EF, M, M), (2, 3, 0, 1))
    return (child_out.reshape(1, M, H),
            sem.reshape(1, M, NS),
            cexl.reshape(1, M, 1),
            eel_out.reshape(1, M, M, ET),
            ef_out.reshape(1, M, M, ET, EF))


# speedup vs baseline: 1.0277x; 1.0277x over previous
import jax
import jax.numpy as jnp
from jax.experimental import pallas as pl
from jax.experimental.pallas import tpu as pltpu

M = 256
H = 128
ED = 128
ET = 4
EF = 4
NS = 57
BS = 32
NBLK = M // BS
CHUNK = 4096
NCH = (H * M) // CHUNK
ROWS = CHUNK // H


def _cf0_kernel(pf_ref, wp_ref, bp_ref, o_ref):
    x = jnp.dot(pf_ref[:, :], wp_ref[:, :], preferred_element_type=jnp.float32)
    o_ref[:, :] = jnp.maximum(x + bp_ref[:, :], 0.0)


NEG = -1e30


def _uv_prologue(cf, wex_ref, bex_ref, wel_ref, bel_ref,
                 u_scr, v_scr, cexl_scr, pend_scr):
    wel = wel_ref[:, :]
    u_scr[:, :] = jnp.dot(cf, wel[:H, :], preferred_element_type=jnp.float32) + bel_ref[:, :]
    v_scr[:, :] = jnp.dot(cf, wel[H:, :], preferred_element_type=jnp.float32)
    cexl_scr[:, :] = jnp.dot(cf, wex_ref[:, :], preferred_element_type=jnp.float32) + bex_ref[:, :]
    pend_scr[:, :] = _pend_col(cexl_scr)


def _masked_reduce(el2, logits2, rhs_ref, cexl_ref, wne, bne,
                   a_scr, b_scr, pend_scr, k):
    wel_i = wne[2 * H:3 * H, :]
    c = jnp.dot(el2, wel_i, preferred_element_type=jnp.float32).reshape(BS, M, H)
    maskpen = (jnp.where(logits2 > 0, jnp.float32(0.0), NEG)
               + pend_scr[:, :])
    lhs = jnp.concatenate([logits2, maskpen], axis=1)
    terms = jnp.dot(lhs, rhs_ref[:, :], preferred_element_type=jnp.float32)
    dmax = jnp.maximum(jnp.maximum(terms[:, :H], terms[:, H:2 * H]),
                       jnp.maximum(terms[:, 2 * H:3 * H], terms[:, 3 * H:]))
    scores = b_scr[:, :][None, :, :] + c + dmax.reshape(BS, M, H)
    d_red = jnp.max(scores, axis=1)
    cexs = cexl_ref[pl.ds(k * BS, BS), :]
    s01 = jnp.where(cexs > 0, jnp.float32(1.0), jnp.float32(0.0))
    a_blk = a_scr[pl.ds(k * BS, BS), :]
    ncf = jnp.maximum(a_blk + bne + d_red, 0.0) * s01
    valid4 = maskpen.reshape(BS, M, ET)
    rowvalid = jnp.max(valid4, axis=(1, 2), keepdims=True).reshape(BS, 1)
    rowflag = jnp.where(jnp.logical_and(rowvalid > NEG * 0.5, cexs > 0),
                        jnp.float32(1.0), jnp.float32(0.0))
    return ncf, rowflag


def _pend_col(cexl_ref):
    pend_m = jnp.where(cexl_ref[:, :] > 0, jnp.float32(0.0), NEG)
    return jnp.broadcast_to(pend_m[None, :, :], (BS, M, 1)).reshape(BS * M, 1)


def _iter1_kernel(wex_ref, bex_ref, wel_ref, bel_ref,
                  wee_ref, bee_ref, bee_t_ref, wef_ref, bef_t_ref,
                  wne_ref, bne_ref, rhs_ref, cf0_ref,
                  cexl_out_ref, eel_ref, ef_ref, ncf_ref, any_ref,
                  a_scr, b_scr, pend_scr, u_scr, v_scr, cexl_scr):
    k = pl.program_id(0)
    wne = wne_ref[:, :]

    @pl.when(k == 0)
    def _():
        cf = cf0_ref[:, :]
        _uv_prologue(cf, wex_ref, bex_ref, wel_ref, bel_ref,
                     u_scr, v_scr, cexl_scr, pend_scr)
        cexl_out_ref[:, :] = cexl_scr[:, :]
        a_scr[:, :] = jnp.dot(cf, wne[:H, :], preferred_element_type=jnp.float32)
        b_scr[:, :] = jnp.dot(cf, wne[H:2 * H, :], preferred_element_type=jnp.float32)
        any_ref[:, :] = jnp.zeros((1, 1), jnp.float32)

    el = jnp.maximum(u_scr[pl.ds(k * BS, BS), :][:, None, :]
                     + v_scr[:, :][None, :, :], 0.0)
    el2 = el.reshape(BS * M, ED)
    logits2 = jnp.dot(el2, wee_ref[:, :], preferred_element_type=jnp.float32) + bee_ref[:, :]
    eelt = jax.lax.dot_general(wee_ref[:, :], el2, (((0,), (1,)), ((), ())),
                               preferred_element_type=jnp.float32)
    eel_ref[:, :] = eelt + bee_t_ref[:, :]
    eft = jax.lax.dot_general(wef_ref[:, :], el2, (((0,), (1,)), ((), ())),
                              preferred_element_type=jnp.float32)
    ef_ref[:, :] = eft + bef_t_ref[:, :]

    ncf, rowflag = _masked_reduce(el2, logits2, rhs_ref, cexl_scr, wne,
                                  bne_ref[:, :], a_scr, b_scr, pend_scr, k)
    ncf_ref[:, :] = ncf
    anyblk = jnp.max(rowflag, axis=(0, 1), keepdims=True)
    any_ref[:, :] = jnp.maximum(any_ref[:, :], anyblk)


def _iter2_kernel(wex_ref, bex_ref, wel_ref, bel_ref, wee_ref, bee_ref,
                  wne_ref, bne_ref, rhs_ref,
                  wc_ref, bc_ref, ws_ref, bs_ref, wc2_ref, bc2_ref,
                  cf0_ref, ncf1_ref, any_ref,
                  sem_ref, out_ref,
                  a_scr, b_scr, pend_scr, u_scr, v_scr, cexl_scr, ncf2_scr):
    k = pl.program_id(0)
    wne = wne_ref[:, :]

    @pl.when(k == 0)
    def _():
        _uv_prologue(cf0_ref[:, :], wex_ref, bex_ref, wel_ref, bel_ref,
                     u_scr, v_scr, cexl_scr, pend_scr)
        anyf = any_ref[:, :]
        cf = ncf1_ref[:, :] * anyf + cf0_ref[:, :] * (1.0 - anyf)
        a_scr[:, :] = jnp.dot(cf, wne[:H, :], preferred_element_type=jnp.float32)
        b_scr[:, :] = jnp.dot(cf, wne[H:2 * H, :], preferred_element_type=jnp.float32)

    el = jnp.maximum(u_scr[pl.ds(k * BS, BS), :][:, None, :]
                     + v_scr[:, :][None, :, :], 0.0)
    el2 = el.reshape(BS * M, ED)
    logits2 = jnp.dot(el2, wee_ref[:, :], preferred_element_type=jnp.float32) + bee_ref[:, :]
    ncf, _ = _masked_reduce(el2, logits2, rhs_ref, cexl_scr, wne,
                            bne_ref[:, :], a_scr, b_scr, pend_scr, k)
    ncf2_scr[pl.ds(k * BS, BS), :] = ncf

    @pl.when(k == NBLK - 1)
    def _():
        anyf = any_ref[:, :]
        cf0 = cf0_ref[:, :]
        cf1 = ncf1_ref[:, :] * anyf + cf0 * (1.0 - anyf)
        cf2 = ncf2_scr[:, :] * anyf + cf0 * (1.0 - anyf)
        cat = jnp.concatenate([cf0, cf1, cf2], axis=1)
        h = jnp.maximum(jnp.dot(cat, wc_ref[:, :], preferred_element_type=jnp.float32)
                        + bc_ref[:, :], 0.0)
        sem_ref[:, :] = jnp.dot(h, ws_ref[:, :], preferred_element_type=jnp.float32) + bs_ref[:, :]
        out_ref[:, :] = jnp.maximum(
            jnp.dot(h, wc2_ref[:, :], preferred_element_type=jnp.float32) + bc2_ref[:, :], 0.0)


def _full(shape):
    return pl.BlockSpec(shape, lambda *_: tuple(0 for _ in shape))


def kernel(parent_feature, Wp, bp, Wex, bex, Wel, bel, Wee, bee, Wne, bne,
           Wc, bc, Ws, bs, Wc2, bc2, Wef, bef):
    f32 = jnp.float32
    bpR = bp.reshape(1, M * H)
    bexR = bex.reshape(1, 1)
    belR = bel.reshape(1, ED)
    WeeR = Wee.reshape(ET, ED).T
    beeR = bee.reshape(1, ET)
    WefR = jnp.transpose(Wef, (1, 0, 2)).reshape(ED, ET * EF)
    befR = bef.reshape(1, ET * EF)
    eye = jnp.eye(ET, dtype=f32)[:, :, None]
    rhs_list = []
    for i in range(2):
        wet = Wne[i, 3 * H:3 * H + ET, :]
        top = (eye * wet[None, :, :]).reshape(ET, ET * H)
        mid = jnp.broadcast_to(eye, (ET, ET, H)).reshape(ET, ET * H)
        rhs_list.append(jnp.concatenate([top, mid], axis=0))

    cf0 = pl.pallas_call(
        _cf0_kernel,
        grid=(NCH,),
        in_specs=[
            pl.BlockSpec((1, H), lambda j: (0, 0)),
            pl.BlockSpec((H, CHUNK), lambda j: (0, j)),
            pl.BlockSpec((1, CHUNK), lambda j: (0, j)),
        ],
        out_specs=pl.BlockSpec((1, CHUNK), lambda j: (0, j)),
        out_shape=jax.ShapeDtypeStruct((1, M * H), f32),
    )(parent_feature, Wp, bpR).reshape(M, H)

    cexl, eelF, efF, ncf1, anyv = pl.pallas_call(
        _iter1_kernel,
        grid=(NBLK,),
        in_specs=[
            _full((H, 1)),
            _full((1, 1)),
            _full((2 * H, ED)),
            _full((1, ED)),
            _full((ED, ET)),
            _full((1, ET)),
            _full((ET, 1)),
            _full((ED, ET * EF)),
            _full((ET * EF, 1)),
            _full((3 * H + ET, H)),
            _full((1, H)),
            _full((2 * ET, ET * H)),
            _full((M, H)),
        ],
        out_specs=[
            pl.BlockSpec((M, 1), lambda k: (0, 0)),
            pl.BlockSpec((ET, BS * M), lambda k: (0, k)),
            pl.BlockSpec((ET * EF, BS * M), lambda k: (0, k)),
            pl.BlockSpec((BS, H), lambda k: (k, 0)),
            pl.BlockSpec((1, 1), lambda k: (0, 0)),
        ],
        out_shape=[jax.ShapeDtypeStruct((M, 1), f32),
                   jax.ShapeDtypeStruct((ET, M * M), f32),
                   jax.ShapeDtypeStruct((ET * EF, M * M), f32),
                   jax.ShapeDtypeStruct((M, H), f32),
                   jax.ShapeDtypeStruct((1, 1), f32)],
        scratch_shapes=[pltpu.VMEM((M, H), f32), pltpu.VMEM((M, H), f32),
                        pltpu.VMEM((BS * M, 1), f32),
                        pltpu.VMEM((M, ED), f32), pltpu.VMEM((M, ED), f32),
                        pltpu.VMEM((M, 1), f32)],
    )(Wex, bexR, Wel, belR, WeeR, beeR, bee.reshape(ET, 1),
      WefR, bef.reshape(ET * EF, 1),
      Wne[0], bne[0].reshape(1, H), rhs_list[0], cf0)

    sem, child_out = pl.pallas_call(
        _iter2_kernel,
        grid=(NBLK,),
        in_specs=[
            _full((H, 1)),
            _full((1, 1)),
            _full((2 * H, ED)),
            _full((1, ED)),
            _full((ED, ET)),
            _full((1, ET)),
            _full((3 * H + ET, H)),
            _full((1, H)),
            _full((2 * ET, ET * H)),
            _full((3 * H, H)),
            _full((1, H)),
            _full((H, NS)),
            _full((1, NS)),
            _full((H, H)),
            _full((1, H)),
            _full((M, H)),
            _full((M, H)),
            _full((1, 1)),
        ],
        out_specs=[
            pl.BlockSpec((M, NS), lambda k: (0, 0)),
            pl.BlockSpec((M, H), lambda k: (0, 0)),
        ],
        out_shape=[jax.ShapeDtypeStruct((M, NS), f32),
                   jax.ShapeDtypeStruct((M, H), f32)],
        scratch_shapes=[pltpu.VMEM((M, H), f32), pltpu.VMEM((M, H), f32),
                        pltpu.VMEM((BS * M, 1), f32),
                        pltpu.VMEM((M, ED), f32), pltpu.VMEM((M, ED), f32),
                        pltpu.VMEM((M, 1), f32), pltpu.VMEM((M, H), f32)],
    )(Wex, bexR, Wel, belR, WeeR, beeR, Wne[1], bne[1].reshape(1, H),
      rhs_list[1], Wc, bc.reshape(1, H), Ws, bs.reshape(1, NS),
      Wc2, bc2.reshape(1, H), cf0, ncf1, anyv)

    eel_out = jnp.transpose(eelF.reshape(ET, M, M), (1, 2, 0))
    ef_out = jnp.transpose(efF.reshape(ET, EF, M, M), (2, 3, 0, 1))
    return (child_out.reshape(1, M, H),
            sem.reshape(1, M, NS),
            cexl.reshape(1, M, 1),
            eel_out.reshape(1, M, M, ET),
            ef_out.reshape(1, M, M, ET, EF))
